# Initial kernel scaffold; baseline (speedup 1.0000x reference)
import jax
import jax.numpy as jnp
from jax.experimental import pallas as pl


def kernel(x, edge_attr, lin0_W, lin0_b, mlp_W1, mlp_b1, mlp_W2, mlp_b2, conv_root, conv_bias, gru_Wih, gru_Whh, gru_bih, gru_bhh, s2s_Wih, s2s_Whh, s2s_bih, s2s_bhh, mem_Wih, mem_Whh, mem_bih, mem_bhh, lin1_W, lin1_b, lin2_W, lin2_b, edge_index, batch, nonring):
    raise NotImplementedError("write your pallas kernel here")


# speedup vs baseline: 1.8054x; 1.8054x over previous
import functools
import jax
import jax.numpy as jnp
from jax import lax
from jax.experimental import pallas as pl
from jax.experimental.pallas import tpu as pltpu
from jax.experimental.pallas import tpu_sc as plsc

NC = 2
NS = 16
NW = NC * NS


def _affine_relu(x, W, b, block):
    M, Kin = x.shape
    Dout = W.shape[0]
    b2 = b.reshape(1, Dout)

    def body(x_ref, w_ref, b_ref, o_ref):
        o_ref[...] = jax.nn.relu(
            jnp.dot(x_ref[...], w_ref[...].T, preferred_element_type=jnp.float32)
            + b_ref[...])

    return pl.pallas_call(
        body,
        grid=(M // block,),
        in_specs=[pl.BlockSpec((block, Kin), lambda i: (i, 0)),
                  pl.BlockSpec((Dout, Kin), lambda i: (0, 0)),
                  pl.BlockSpec((1, Dout), lambda i: (0, 0))],
        out_specs=pl.BlockSpec((block, Dout), lambda i: (i, 0)),
        out_shape=jax.ShapeDtypeStruct((M, Dout), jnp.float32),
    )(x, W, b2)


def _msg_kernel(xj, he, M2s, B2m, block):
    E, D = xj.shape

    def body(xj_ref, he_ref, m2_ref, b2_ref, o_ref):
        xjv = xj_ref[...]
        hev = he_ref[...]
        acc = jnp.dot(xjv, b2_ref[...], preferred_element_type=jnp.float32)
        for i in range(D):
            G = jnp.dot(hev, m2_ref[i], preferred_element_type=jnp.float32)
            acc = acc + xjv[:, i:i + 1] * G
        o_ref[...] = acc

    return pl.pallas_call(
        body,
        grid=(E // block,),
        in_specs=[pl.BlockSpec((block, D), lambda i: (i, 0)),
                  pl.BlockSpec((block, D), lambda i: (i, 0)),
                  pl.BlockSpec((D, D, D), lambda i: (0, 0, 0)),
                  pl.BlockSpec((D, D), lambda i: (0, 0))],
        out_specs=pl.BlockSpec((block, D), lambda i: (i, 0)),
        out_shape=jax.ShapeDtypeStruct((E, D), jnp.float32),
    )(xj, he, M2s, B2m)


def _update_kernel(aggr_p, cnt_p, out, conv_root, conv_bias,
                   gru_Wih, gru_Whh, gru_bih, gru_bhh, block):
    N, D = out.shape

    def body(ap_ref, cp_ref, out_ref, root_ref, cb_ref,
             wih_ref, whh_ref, bih_ref, bhh_ref, o_ref):
        aggr = ap_ref[0] + ap_ref[1]
        cnt = jnp.clip(cp_ref[0] + cp_ref[1], 1.0, None)
        outv = out_ref[...]
        m = jax.nn.relu(
            aggr / cnt
            + jnp.dot(outv, root_ref[...], preferred_element_type=jnp.float32)
            + cb_ref[...])
        gi = jnp.dot(m, wih_ref[...].T, preferred_element_type=jnp.float32) + bih_ref[...]
        gh = jnp.dot(outv, whh_ref[...].T, preferred_element_type=jnp.float32) + bhh_ref[...]
        r = jax.nn.sigmoid(gi[:, :D] + gh[:, :D])
        z = jax.nn.sigmoid(gi[:, D:2 * D] + gh[:, D:2 * D])
        ng = jnp.tanh(gi[:, 2 * D:] + r * gh[:, 2 * D:])
        o_ref[...] = (1.0 - z) * ng + z * outv

    return pl.pallas_call(
        body,
        grid=(N // block,),
        in_specs=[pl.BlockSpec((2, block, D), lambda i: (0, i, 0)),
                  pl.BlockSpec((2, block, D), lambda i: (0, i, 0)),
                  pl.BlockSpec((block, D), lambda i: (i, 0)),
                  pl.BlockSpec((D, D), lambda i: (0, 0)),
                  pl.BlockSpec((1, D), lambda i: (0, 0)),
                  pl.BlockSpec((3 * D, D), lambda i: (0, 0)),
                  pl.BlockSpec((3 * D, D), lambda i: (0, 0)),
                  pl.BlockSpec((1, 3 * D), lambda i: (0, 0)),
                  pl.BlockSpec((1, 3 * D), lambda i: (0, 0))],
        out_specs=pl.BlockSpec((block, D), lambda i: (i, 0)),
        out_shape=jax.ShapeDtypeStruct((N, D), jnp.float32),
    )(aggr_p, cnt_p, out, conv_root, conv_bias.reshape(1, D),
      gru_Wih, gru_Whh, gru_bih.reshape(1, 3 * D), gru_bhh.reshape(1, 3 * D))


def _set2set_kernel(out, s2s_Wih, s2s_Whh, s2s_bih, s2s_bhh,
                    mem_Wih, mem_Whh, mem_bih, mem_bhh):
    N, D = out.shape

    def body(out_ref, swih_ref, swhh_ref, sbih_ref, sbhh_ref,
             mwih_ref, mwhh_ref, mbih_ref, mbhh_ref, hx_ref, cx_ref):
        outv = out_ref[...]
        q_star = jnp.zeros((1, 2 * D), jnp.float32)
        hh = jnp.zeros((1, D), jnp.float32)
        cc = jnp.zeros((1, D), jnp.float32)
        for _ in range(6):
            g = (jnp.dot(q_star, swih_ref[...].T, preferred_element_type=jnp.float32)
                 + sbih_ref[...]
                 + jnp.dot(hh, swhh_ref[...].T, preferred_element_type=jnp.float32)
                 + sbhh_ref[...])
            ig = g[:, :D]
            fg = g[:, D:2 * D]
            gg = g[:, 2 * D:3 * D]
            og = g[:, 3 * D:]
            cc = jax.nn.sigmoid(fg) * cc + jax.nn.sigmoid(ig) * jnp.tanh(gg)
            hh = jax.nn.sigmoid(og) * jnp.tanh(cc)
            q = hh
            e = jnp.sum(outv * q, axis=1, keepdims=True)
            a = jnp.exp(e - jnp.max(e))
            a = a / jnp.sum(a)
            r_t = jnp.sum(a * outv, axis=0, keepdims=True)
            q_star = jnp.concatenate([q, r_t], axis=1)
        g = (jnp.dot(q_star, mwih_ref[...].T, preferred_element_type=jnp.float32)
             + mbih_ref[...] + mbhh_ref[...])
        ig = g[:, :D]
        fg = g[:, D:2 * D]
        gg = g[:, 2 * D:3 * D]
        og = g[:, 3 * D:]
        cx = jax.nn.sigmoid(fg) * 0.0 + jax.nn.sigmoid(ig) * jnp.tanh(gg)
        hx = jax.nn.sigmoid(og) * jnp.tanh(cx)
        hx_ref[...] = hx
        cx_ref[...] = cx

    full = lambda s: pl.BlockSpec(s, lambda: tuple(0 for _ in s))
    return pl.pallas_call(
        body,
        in_specs=[full((N, D)), full((4 * D, 2 * D)), full((4 * D, D)),
                  full((1, 4 * D)), full((1, 4 * D)),
                  full((4 * D, 2 * D)), full((4 * D, D)),
                  full((1, 4 * D)), full((1, 4 * D))],
        out_specs=[full((1, D)), full((1, D))],
        out_shape=[jax.ShapeDtypeStruct((1, D), jnp.float32),
                   jax.ShapeDtypeStruct((1, D), jnp.float32)],
    )(out, s2s_Wih, s2s_Whh, s2s_bih.reshape(1, -1), s2s_bhh.reshape(1, -1),
      mem_Wih, mem_Whh, mem_bih.reshape(1, -1), mem_bhh.reshape(1, -1))


def _head_kernel(zc, lin1_W, lin1_b, lin2_W, lin2_b):
    K, F = zc.shape
    D = lin1_W.shape[0]
    A = lin2_W.shape[0]

    def body(z_ref, w1_ref, b1_ref, w2_ref, b2_ref, o_ref):
        h = jax.nn.relu(
            jnp.dot(z_ref[...], w1_ref[...].T, preferred_element_type=jnp.float32)
            + b1_ref[...])
        o_ref[...] = (jnp.dot(h, w2_ref[...].T, preferred_element_type=jnp.float32)
                      + b2_ref[...])

    full = lambda s: pl.BlockSpec(s, lambda: tuple(0 for _ in s))
    return pl.pallas_call(
        body,
        in_specs=[full((K, F)), full((D, F)), full((1, D)),
                  full((A, D)), full((1, A))],
        out_specs=full((K, A)),
        out_shape=jax.ShapeDtypeStruct((K, A), jnp.float32),
    )(zc, lin1_W, lin1_b.reshape(1, D), lin2_W, lin2_b.reshape(1, A))


def _sc_gather(table, idx):
    B = idx.shape[0]
    V, D = table.shape
    per_w = B // NW
    nfull = per_w // 128
    rem = per_w % 128
    mesh = plsc.VectorSubcoreMesh(core_axis_name="c", subcore_axis_name="s")

    scratch = [pltpu.VMEM((128,), jnp.int32),
               pltpu.VMEM((128, D), jnp.float32)]
    if rem:
        scratch += [pltpu.VMEM((rem,), jnp.int32),
                    pltpu.VMEM((rem, D), jnp.float32)]
    scratch += [pltpu.SemaphoreType.DMA]

    @functools.partial(
        pl.kernel, mesh=mesh,
        out_type=jax.ShapeDtypeStruct((B, D), jnp.float32),
        scratch_types=scratch,
        compiler_params=pltpu.CompilerParams(use_tc_tiling_on_sc=False))
    def k(table_hbm, idx_hbm, out_hbm, *sc):
        if rem:
            idx_v, rows_v, idx_v2, rows_v2, sem = sc
        else:
            idx_v, rows_v, sem = sc
        wid = lax.axis_index("s") * NC + lax.axis_index("c")
        base = wid * per_w

        def step(j, carry):
            off = base + j * 128
            pltpu.sync_copy(idx_hbm.at[pl.ds(off, 128)], idx_v)
            pltpu.async_copy(table_hbm.at[idx_v], rows_v, sem).wait()
            pltpu.sync_copy(rows_v, out_hbm.at[pl.ds(off, 128)])
            return carry

        if nfull:
            lax.fori_loop(0, nfull, step, 0)
        if rem:
            off = base + nfull * 128
            pltpu.sync_copy(idx_hbm.at[pl.ds(off, rem)], idx_v2)
            pltpu.async_copy(table_hbm.at[idx_v2], rows_v2, sem).wait()
            pltpu.sync_copy(rows_v2, out_hbm.at[pl.ds(off, rem)])

    return k(table, idx)


def _sc_scatter_add(msg, dst, zeros_nd):
    E, D = msg.shape
    N = zeros_nd.shape[0]
    E2 = E // NC
    per_w = E2 // NS
    nfull = per_w // 128
    rem = per_w % 128
    mesh = plsc.VectorSubcoreMesh(core_axis_name="c", subcore_axis_name="s")

    scratch = [pltpu.VMEM((128,), jnp.int32),
               pltpu.VMEM((128, D), jnp.float32)]
    if rem:
        scratch += [pltpu.VMEM((rem,), jnp.int32),
                    pltpu.VMEM((rem, D), jnp.float32)]
    scratch += [pltpu.VMEM_SHARED((N, D), jnp.float32)]

    @functools.partial(
        pl.kernel, mesh=mesh,
        out_type=jax.ShapeDtypeStruct((NC, N, D), jnp.float32),
        scratch_types=scratch,
        compiler_params=pltpu.CompilerParams(use_tc_tiling_on_sc=False))
    def k(msg_hbm, dst_hbm, zero_hbm, out_hbm, *sc):
        if rem:
            idx_v, rows_v, idx_v2, rows_v2, shared = sc
        else:
            idx_v, rows_v, shared = sc
        c = lax.axis_index("c")
        s = lax.axis_index("s")

        @pl.when(s == 0)
        def _():
            pltpu.sync_copy(zero_hbm, shared)

        plsc.subcore_barrier()
        base = c * E2 + s * per_w

        def step(j, carry):
            off = base + j * 128
            pltpu.sync_copy(dst_hbm.at[pl.ds(off, 128)], idx_v)
            pltpu.sync_copy(msg_hbm.at[pl.ds(off, 128)], rows_v)
            pltpu.sync_copy(rows_v, shared.at[idx_v], add=True)
            return carry

        if nfull:
            lax.fori_loop(0, nfull, step, 0)
        if rem:
            off = base + nfull * 128
            pltpu.sync_copy(dst_hbm.at[pl.ds(off, rem)], idx_v2)
            pltpu.sync_copy(msg_hbm.at[pl.ds(off, rem)], rows_v2)
            pltpu.sync_copy(rows_v2, shared.at[idx_v2], add=True)

        plsc.subcore_barrier()

        @pl.when(s == 0)
        def _():
            pltpu.sync_copy(shared, out_hbm.at[c])

    return k(msg, dst, zeros_nd)


@jax.jit
def kernel(x, edge_attr, lin0_W, lin0_b, mlp_W1, mlp_b1, mlp_W2, mlp_b2,
           conv_root, conv_bias, gru_Wih, gru_Whh, gru_bih, gru_bhh,
           s2s_Wih, s2s_Whh, s2s_bih, s2s_bhh, mem_Wih, mem_Whh, mem_bih,
           mem_bhh, lin1_W, lin1_b, lin2_W, lin2_b, edge_index, batch, nonring):
    N, _ = x.shape
    E = edge_attr.shape[0]
    D = lin0_W.shape[0]
    src = edge_index[0]
    dst = edge_index[1]

    W2r = mlp_W2.reshape(D, D, D)
    M2s = W2r.transpose(0, 2, 1)
    B2m = mlp_b2.reshape(D, D)
    zeros_nd = jnp.zeros((N, D), jnp.float32)
    ones_e = jnp.ones((E, D), jnp.float32)

    out = _affine_relu(x, lin0_W, lin0_b, block=2000)
    he = _affine_relu(edge_attr, mlp_W1, mlp_b1, block=4000)

    cnt_p = _sc_scatter_add(ones_e, dst, zeros_nd)

    for _ in range(6):
        xj = _sc_gather(out, src)
        msg = _msg_kernel(xj, he, M2s, B2m, block=2000)
        aggr_p = _sc_scatter_add(msg, dst, zeros_nd)
        out = _update_kernel(aggr_p, cnt_p, out, conv_root, conv_bias,
                             gru_Wih, gru_Whh, gru_bih, gru_bhh, block=2000)

    hx, cx = _set2set_kernel(out, s2s_Wih, s2s_Whh, s2s_bih, s2s_bhh,
                             mem_Wih, mem_Whh, mem_bih, mem_bhh)

    sel = _sc_gather(out, nonring.reshape(-1))
    sel = sel.reshape(4 * D, -1).T
    rep = jnp.repeat(hx.reshape(-1), sel.shape[0]).reshape(sel.shape[0], -1)
    zc = jnp.concatenate([sel, rep], axis=1)
    zc = _head_kernel(zc, lin1_W, lin1_b, lin2_W, lin2_b)
    return zc, hx, cx
